# baseline (speedup 1.0000x reference)
import functools

import jax
import jax.numpy as jnp
from jax import lax
from jax.experimental import pallas as pl
from jax.experimental.pallas import tpu as pltpu
from jax.experimental.pallas import tpu_sc as plsc

N_NODES = 10000
N_GRAPHS = 16
N_MOM = 4
F = 128
LANES = 16
NJ = F // LANES
NB = 250
NCHUNK = N_NODES // NB
NBI = N_NODES // LANES
NACC = NJ * 2
ACC_LEN = N_GRAPHS * N_MOM * F


def _sc_body(x_hbm, bi_hbm, out_hbm, bi_v, buf0, buf1, acc, sem0, sem1, bi_sem):
    cid = lax.axis_index("c")
    sid = lax.axis_index("s")
    wid = cid * 16 + sid
    o = wid // 16
    km = wid % 16

    zf = jnp.zeros((LANES,), jnp.float32)

    def start_dma(chunk, buf, sem):
        src = x_hbm.at[o, pl.ds(chunk * NB, NB), km]
        pltpu.make_async_copy(src, buf.at[pl.ds(0, NB)], sem).start()

    def wait_dma(chunk, buf, sem):
        src = x_hbm.at[o, pl.ds(chunk * NB, NB), km]
        pltpu.make_async_copy(src, buf.at[pl.ds(0, NB)], sem).wait()

    start_dma(jnp.int32(0), buf0, sem0)
    start_dma(jnp.int32(1), buf1, sem1)

    pltpu.make_async_copy(bi_hbm, bi_v, bi_sem).start()

    def zero_body(i, _):
        acc[pl.ds(i * LANES, LANES)] = zf
        return 0

    lax.fori_loop(0, ACC_LEN // LANES, zero_body, 0, unroll=False)

    pltpu.make_async_copy(bi_hbm, bi_v, bi_sem).wait()

    def seg_start(g):
        lo = jnp.int32(0)
        hi = jnp.int32(NBI)
        for _ in range(10):
            mid = lax.div(lo + hi, jnp.int32(2))
            midc = jnp.minimum(mid, jnp.int32(NBI - 1))
            v = bi_v[pl.ds(midc * LANES, LANES)]
            pred = (v[0] < g) & (lo < hi)
            lo = jnp.where(pred, mid + 1, lo)
            hi = jnp.where(pred, hi, mid)
        cm1 = jnp.maximum(lo - 1, jnp.int32(0))
        ch = bi_v[pl.ds(cm1 * LANES, LANES)]
        cnt = jnp.int32(0)
        for i in range(LANES):
            cnt = cnt + jnp.where(ch[i] < g, jnp.int32(1), jnp.int32(0))
        return cm1 * LANES + cnt

    starts = [jnp.int32(0)]
    for g in range(1, N_GRAPHS):
        starts.append(seg_start(jnp.int32(g)))
    starts.append(jnp.int32(N_NODES))

    def flush(cur_g, accs, negate=False):
        goff = cur_g * (N_MOM * F)
        for j in range(NJ):
            for s in range(2):
                off = goff + s * F + j * LANES
                v = accs[j * 2 + s]
                plsc.addupdate(acc.at[pl.ds(off, LANES)], -v if negate else v)

    def advance(p):
        ng = jnp.int32(0)
        for gg in range(1, N_GRAPHS):
            ng = jnp.where(starts[gg] <= p, jnp.int32(gg), ng)
        nb = jnp.int32(N_NODES)
        for gg in range(N_GRAPHS - 1, 0, -1):
            nb = jnp.where(ng + 1 == gg, starts[gg], nb)
        return ng, nb

    def load_node(buf, n):
        return tuple(buf[n, pl.ds(j * LANES, LANES)] for j in range(NJ))

    def moment_update(accs, data, goff):
        for j in range(NJ):
            a = jnp.abs(data[j])
            a2 = data[j] * data[j]
            a3 = a2 * a
            a4 = a2 * a2
            k2 = j * 2
            accs[k2] = accs[k2] + a
            accs[k2 + 1] = accs[k2 + 1] + a2
            plsc.addupdate(acc.at[pl.ds(goff + 2 * F + j * LANES, LANES)], a3)
            plsc.addupdate(acc.at[pl.ds(goff + 3 * F + j * LANES, LANES)], a4)
        return accs

    def process_chunk(chunk, buf, carry):
        base = chunk * NB

        def node_body(n, carry):
            accs = list(carry[0])
            data = carry[1]
            cur_g, next_b = carry[2], carry[3]
            p = base + n

            def on_flush(c):
                cur_g, _ = c
                flush(cur_g, accs)
                ng, nb = advance(p)
                flush(ng, accs, negate=True)
                return (ng, nb)

            def no_flush(c):
                return c

            cur_g, next_b = lax.cond(
                p >= next_b, on_flush, no_flush, (cur_g, next_b)
            )

            accs = moment_update(accs, data, cur_g * (N_MOM * F))
            ndata = load_node(buf, n + 1)
            return (tuple(accs), ndata, cur_g, next_b)

        c = lax.fori_loop(
            0, NB, node_body,
            (carry[0], load_node(buf, 0), carry[1], carry[2]),
            unroll=5,
        )
        return (c[0], c[2], c[3])

    def outer(ci, carry):
        c = ci * 2
        wait_dma(c, buf0, sem0)
        carry = process_chunk(c, buf0, carry)

        @pl.when(c + 2 < NCHUNK)
        def _():
            start_dma(c + 2, buf0, sem0)

        wait_dma(c + 1, buf1, sem1)
        carry = process_chunk(c + 1, buf1, carry)

        @pl.when(c + 3 < NCHUNK)
        def _():
            start_dma(c + 3, buf1, sem1)

        return carry

    g0, nb0 = advance(jnp.int32(0))
    init = (tuple(zf for _ in range(NACC)), g0, nb0)
    accs, cur_g, _ = lax.fori_loop(0, NCHUNK // 2, outer, init, unroll=False)
    flush(cur_g, list(accs))

    pltpu.sync_copy(acc, out_hbm.at[wid])


@jax.jit
def kernel(x, batch_index):
    x4 = x.reshape(2, N_NODES, 16, F)
    bi = batch_index.astype(jnp.int32)

    mesh = plsc.VectorSubcoreMesh(core_axis_name="c", subcore_axis_name="s")
    run = pl.kernel(
        _sc_body,
        out_type=jax.ShapeDtypeStruct((32, ACC_LEN), jnp.float32),
        mesh=mesh,
        scratch_types=[
            pltpu.VMEM((N_NODES,), jnp.int32),
            pltpu.VMEM((NB + 1, F), jnp.float32),
            pltpu.VMEM((NB + 1, F), jnp.float32),
            pltpu.VMEM((ACC_LEN,), jnp.float32),
            pltpu.SemaphoreType.DMA,
            pltpu.SemaphoreType.DMA,
            pltpu.SemaphoreType.DMA,
        ],
    )
    out32 = run(x4, bi)
    out = out32.reshape(2, 4, 4, N_GRAPHS, N_MOM, F)
    return jnp.transpose(out, (3, 0, 4, 1, 2, 5))


# speedup vs baseline: 1.2564x; 1.2564x over previous
import functools

import jax
import jax.numpy as jnp
from jax import lax
from jax.experimental import pallas as pl
from jax.experimental.pallas import tpu as pltpu
from jax.experimental.pallas import tpu_sc as plsc

N_NODES = 10000
N_GRAPHS = 16
N_MOM = 4
F = 128
LANES = 16
NJ = F // LANES
NB = 250
NCHUNK = N_NODES // NB
NBI = N_NODES // LANES
NACC = NJ * 3
ACC_LEN = N_GRAPHS * N_MOM * F


def _sc_body(x_hbm, bi_hbm, out_hbm, bi_v, buf0, buf1, acc, sem0, sem1, bi_sem):
    cid = lax.axis_index("c")
    sid = lax.axis_index("s")
    wid = cid * 16 + sid
    o = wid // 16
    km = wid % 16

    zf = jnp.zeros((LANES,), jnp.float32)

    def start_dma(chunk, buf, sem):
        src = x_hbm.at[o, pl.ds(chunk * NB, NB), km]
        pltpu.make_async_copy(src, buf.at[pl.ds(0, NB)], sem).start()

    def wait_dma(chunk, buf, sem):
        src = x_hbm.at[o, pl.ds(chunk * NB, NB), km]
        pltpu.make_async_copy(src, buf.at[pl.ds(0, NB)], sem).wait()

    start_dma(jnp.int32(0), buf0, sem0)
    start_dma(jnp.int32(1), buf1, sem1)

    pltpu.make_async_copy(bi_hbm, bi_v, bi_sem).start()

    def zero_body(i, _):
        acc[pl.ds(i * LANES, LANES)] = zf
        return 0

    lax.fori_loop(0, ACC_LEN // LANES, zero_body, 0, unroll=False)

    pltpu.make_async_copy(bi_hbm, bi_v, bi_sem).wait()

    def seg_start(g):
        lo = jnp.int32(0)
        hi = jnp.int32(NBI)
        for _ in range(10):
            mid = lax.div(lo + hi, jnp.int32(2))
            midc = jnp.minimum(mid, jnp.int32(NBI - 1))
            v = bi_v[pl.ds(midc * LANES, LANES)]
            pred = (v[0] < g) & (lo < hi)
            lo = jnp.where(pred, mid + 1, lo)
            hi = jnp.where(pred, hi, mid)
        cm1 = jnp.maximum(lo - 1, jnp.int32(0))
        ch = bi_v[pl.ds(cm1 * LANES, LANES)]
        cnt = jnp.int32(0)
        for i in range(LANES):
            cnt = cnt + jnp.where(ch[i] < g, jnp.int32(1), jnp.int32(0))
        return cm1 * LANES + cnt

    starts = [jnp.int32(0)]
    for g in range(1, N_GRAPHS):
        starts.append(seg_start(jnp.int32(g)))
    starts.append(jnp.int32(N_NODES))

    def flush(cur_g, accs, negate=False):
        goff = cur_g * (N_MOM * F)
        for j in range(NJ):
            for s in range(3):
                off = goff + s * F + j * LANES
                v = accs[j * 3 + s]
                plsc.addupdate(acc.at[pl.ds(off, LANES)], -v if negate else v)

    def advance(p):
        ng = jnp.int32(0)
        for gg in range(1, N_GRAPHS):
            ng = jnp.where(starts[gg] <= p, jnp.int32(gg), ng)
        nb = jnp.int32(N_NODES)
        for gg in range(N_GRAPHS - 1, 0, -1):
            nb = jnp.where(ng + 1 == gg, starts[gg], nb)
        return ng, nb

    def load_node(buf, n):
        return tuple(buf[n, pl.ds(j * LANES, LANES)] for j in range(NJ))

    def moment_update(accs, data, goff):
        for j in range(NJ):
            a = jnp.abs(data[j])
            a2 = data[j] * data[j]
            a3 = a2 * a
            a4 = a2 * a2
            k3 = j * 3
            accs[k3] = accs[k3] + a
            accs[k3 + 1] = accs[k3 + 1] + a2
            accs[k3 + 2] = accs[k3 + 2] + a3
            plsc.addupdate(acc.at[pl.ds(goff + 3 * F + j * LANES, LANES)], a4)
        return accs

    def process_chunk(chunk, buf, carry):
        base = chunk * NB

        def node_body(n, carry):
            accs = list(carry[0])
            data = carry[1]
            cur_g, next_b = carry[2], carry[3]
            p = base + n

            def on_flush(c):
                cur_g, _ = c
                flush(cur_g, accs)
                ng, nb = advance(p)
                flush(ng, accs, negate=True)
                return (ng, nb)

            def no_flush(c):
                return c

            cur_g, next_b = lax.cond(
                p >= next_b, on_flush, no_flush, (cur_g, next_b)
            )

            accs = moment_update(accs, data, cur_g * (N_MOM * F))
            ndata = load_node(buf, n + 1)
            return (tuple(accs), ndata, cur_g, next_b)

        c = lax.fori_loop(
            0, NB, node_body,
            (carry[0], load_node(buf, 0), carry[1], carry[2]),
            unroll=10,
        )
        return (c[0], c[2], c[3])

    def outer(ci, carry):
        c = ci * 2
        wait_dma(c, buf0, sem0)
        carry = process_chunk(c, buf0, carry)

        @pl.when(c + 2 < NCHUNK)
        def _():
            start_dma(c + 2, buf0, sem0)

        wait_dma(c + 1, buf1, sem1)
        carry = process_chunk(c + 1, buf1, carry)

        @pl.when(c + 3 < NCHUNK)
        def _():
            start_dma(c + 3, buf1, sem1)

        return carry

    g0, nb0 = advance(jnp.int32(0))
    init = (tuple(zf for _ in range(NACC)), g0, nb0)
    accs, cur_g, _ = lax.fori_loop(0, NCHUNK // 2, outer, init, unroll=False)
    flush(cur_g, list(accs))

    pltpu.sync_copy(acc, out_hbm.at[wid])


@jax.jit
def kernel(x, batch_index):
    x4 = x.reshape(2, N_NODES, 16, F)
    bi = batch_index.astype(jnp.int32)

    mesh = plsc.VectorSubcoreMesh(core_axis_name="c", subcore_axis_name="s")
    run = pl.kernel(
        _sc_body,
        out_type=jax.ShapeDtypeStruct((32, ACC_LEN), jnp.float32),
        mesh=mesh,
        scratch_types=[
            pltpu.VMEM((N_NODES,), jnp.int32),
            pltpu.VMEM((NB + 1, F), jnp.float32),
            pltpu.VMEM((NB + 1, F), jnp.float32),
            pltpu.VMEM((ACC_LEN,), jnp.float32),
            pltpu.SemaphoreType.DMA,
            pltpu.SemaphoreType.DMA,
            pltpu.SemaphoreType.DMA,
        ],
    )
    out32 = run(x4, bi)
    out = out32.reshape(2, 4, 4, N_GRAPHS, N_MOM, F)
    return jnp.transpose(out, (3, 0, 4, 1, 2, 5))
